# Initial kernel scaffold; baseline (speedup 1.0000x reference)
import jax
import jax.numpy as jnp
from jax.experimental import pallas as pl


def kernel(x, edge_index, edge_weight, W1, b1, W2, b2, W3, b3, g1, bt1, g2, bt2, g3, bt3):
    raise NotImplementedError("write your pallas kernel here")


# speedup vs baseline: 11.2737x; 11.2737x over previous
import functools

import jax
import jax.numpy as jnp
from jax import lax
from jax.experimental import pallas as pl
from jax.experimental.pallas import tpu as pltpu
from jax.experimental.pallas import tpu_sc as plsc

N = 10000
E = 320000
NC = 2
NS = 16
NW = NC * NS
EPT = E // NW
BLK = 400
SUB = 80
NSUB = BLK // SUB
NBLK = EPT // BLK
NPT = N // NS
F32 = jnp.float32
I32 = jnp.int32

_MESH = plsc.VectorSubcoreMesh(
    core_axis_name="c", subcore_axis_name="s", num_cores=NC, num_subcores=NS)
_SC_PARAMS = pltpu.CompilerParams(use_tc_tiling_on_sc=False)


def _zero_rows(rows, nrows, ncols):
    def zr(i, _):
        for c0 in range(0, ncols, 16):
            rows[i, pl.ds(c0, 16)] = jnp.zeros((16,), F32)
        return 0
    lax.fori_loop(0, nrows, zr, 0)


_GDN = lax.GatherDimensionNumbers(
    offset_dims=(), collapsed_slice_dims=(0,), start_index_map=(0,))


def _bcast(v16, lane):
    idx = jnp.full((16, 1), lane, I32)
    return lax.gather(v16, idx, _GDN, (1,),
                      mode=lax.GatherScatterMode.PROMISE_IN_BOUNDS)


def _zero_acc(rows, acc, s):
    base = s * NPT
    pltpu.sync_copy(rows.at[pl.ds(0, BLK)], acc.at[pl.ds(base, BLK)])
    pltpu.sync_copy(rows.at[pl.ds(0, NPT - BLK)],
                    acc.at[pl.ds(base + BLK, NPT - BLK)])


def _make_sc_pass(C):
    @functools.partial(
        pl.kernel,
        out_type=jax.ShapeDtypeStruct((NC, NS, NPT, C), F32),
        mesh=_MESH,
        compiler_params=_SC_PARAMS,
        scratch_types=[
            pltpu.VMEM((BLK, C), F32),
            pltpu.VMEM((NSUB, SUB), I32),
            pltpu.VMEM((NSUB, SUB), I32),
            pltpu.VMEM((BLK,), F32),
            pltpu.VMEM_SHARED((N, C), F32),
            pltpu.SemaphoreType.DMA,
            pltpu.SemaphoreType.DMA,
        ],
    )
    def _sc_pass(tab_hbm, src_hbm, dst_hbm, ew_hbm, out_hbm,
                 rows, srcv, dstv, eww, acc, gsem, ssem):
        c = lax.axis_index("c")
        s = lax.axis_index("s")
        wid = s * NC + c
        _zero_rows(rows, BLK, C)
        _zero_acc(rows, acc, s)
        plsc.subcore_barrier()

        def blk(i, _):
            ebase = wid * EPT + i * BLK
            pltpu.sync_copy(src_hbm.at[wid, i], srcv)
            pltpu.sync_copy(dst_hbm.at[wid, i], dstv)
            pltpu.sync_copy(ew_hbm.at[pl.ds(ebase, BLK)], eww)
            gcps = [
                pltpu.async_copy(tab_hbm.at[srcv.at[j]],
                                 rows.at[pl.ds(j * SUB, SUB)], gsem)
                for j in range(NSUB)
            ]
            for cp in gcps:
                cp.wait()

            def grp(g, _):
                e0 = g * 16
                w16 = eww[pl.ds(e0, 16)]
                for l in range(16):
                    wb = _bcast(w16, l)
                    for c0 in range(0, C, 16):
                        v = rows[e0 + l, pl.ds(c0, 16)]
                        rows[e0 + l, pl.ds(c0, 16)] = v * wb
                return 0
            lax.fori_loop(0, BLK // 16, grp, 0)
            for j in range(NSUB):
                pltpu.sync_copy(rows.at[pl.ds(j * SUB, SUB)],
                                acc.at[dstv.at[j]], add=True)
            return 0
        lax.fori_loop(0, NBLK, blk, 0)
        plsc.subcore_barrier()
        pltpu.sync_copy(acc.at[pl.ds(s * NPT, NPT)], out_hbm.at[c, s])
    return _sc_pass


_sc_pass64 = _make_sc_pass(64)
_sc_pass32 = _make_sc_pass(32)


def _proj1_body(degp, x, wab, wz, dis_o, dis2_o, ab_o, z0_o):
    deg = degp[0, :, 0:1] + degp[1, :, 0:1]
    dis = jnp.where(deg > 0, lax.rsqrt(jnp.where(deg > 0, deg, 1.0)), 0.0)
    dis_o[...] = dis
    dis2_o[...] = dis * dis
    xx = x[...]
    ab_o[...] = jnp.dot(xx, wab[...], preferred_element_type=F32) * dis
    z0_o[...] = jnp.dot(xx, wz[...], preferred_element_type=F32)


def _tc_proj1(degp, x2, wab, wz):
    return pl.pallas_call(
        _proj1_body,
        out_shape=[
            jax.ShapeDtypeStruct((N, 1), F32),
            jax.ShapeDtypeStruct((N, 1), F32),
            jax.ShapeDtypeStruct((N, 64), F32),
            jax.ShapeDtypeStruct((N, 32), F32),
        ],
    )(degp, x2, wab, wz)


def _make_mid(c0):
    def _mid_body(parts, dis2, out):
        v = parts[0, :, c0:c0 + 32] + parts[1, :, c0:c0 + 32]
        out[...] = v * dis2[...]

    def _mid(parts, dis2):
        return pl.pallas_call(
            _mid_body,
            out_shape=jax.ShapeDtypeStruct((N, 32), F32),
        )(parts, dis2)
    return _mid


_tc_mid64 = _make_mid(32)
_tc_mid32 = _make_mid(0)


def _bn_relu(ypre, g, bt):
    m = jnp.mean(ypre, axis=0, keepdims=True)
    var = jnp.mean((ypre - m) ** 2, axis=0, keepdims=True)
    y = (ypre - m) * lax.rsqrt(var + 1e-5) * g + bt
    return jnp.maximum(y, 0.0)


def _comb1_body(z0, up, pp, dis_r, b1, g1, bt1, y1_o, s3_o):
    dis = dis_r[...]
    u = up[0, :, 0:32] + up[1, :, 0:32]
    p = pp[0] + pp[1]
    ypre = z0[...] - dis * u + 2.0 * dis * p + b1[...]
    y = _bn_relu(ypre, g1[...], bt1[...])
    y1_o[...] = y
    s3_o[...] = y * dis


def _tc_comb1(z0, up, pp, dis, b1, g1, bt1):
    return pl.pallas_call(
        _comb1_body,
        out_shape=[
            jax.ShapeDtypeStruct((N, 32), F32),
            jax.ShapeDtypeStruct((N, 32), F32),
        ],
    )(z0, up, pp, dis, b1, g1, bt1)


def _make_comb(cout):
    def _comb_body(y_in, qp, rp, dis_r, wc, b, g, bt, y_o, s_o):
        dis = dis_r[...]
        q = qp[0] + qp[1]
        r = rp[0] + rp[1]
        h = jnp.concatenate([y_in[...], dis * q, dis * r], axis=1)
        ypre = jnp.dot(h, wc[...], preferred_element_type=F32) + b[...]
        y = _bn_relu(ypre, g[...], bt[...])
        y_o[...] = y
        s_o[...] = (y * dis) if cout == 32 else jnp.zeros((N, 1), F32)

    def _comb(y_in, qp, rp, dis, wc, b, g, bt):
        return pl.pallas_call(
            _comb_body,
            out_shape=[
                jax.ShapeDtypeStruct((N, cout), F32),
                jax.ShapeDtypeStruct((N, 32 if cout == 32 else 1), F32),
            ],
        )(y_in, qp, rp, dis, wc, b, g, bt)
    return _comb


_tc_comb2 = _make_comb(32)


def _comb3_body(x, y2, sp, tp, dis_r, wc, b, g, bt, out_o):
    dis = dis_r[...]
    s = sp[0] + sp[1]
    t = tp[0] + tp[1]
    h = jnp.concatenate([y2[...], dis * s, dis * t], axis=1)
    ypre = jnp.dot(h, wc[...], preferred_element_type=F32) + b[...]
    y = _bn_relu(ypre, g[...], bt[...])
    out_o[...] = jnp.maximum(x[...] + y, 0.0)


def _tc_comb3(x2, y2, sp, tp, dis, wc, b, g, bt):
    return pl.pallas_call(
        _comb3_body,
        out_shape=jax.ShapeDtypeStruct((N, 128), F32),
    )(x2, y2, sp, tp, dis, wc, b, g, bt)


def kernel(x, edge_index, edge_weight, W1, b1, W2, b2, W3, b3,
           g1, bt1, g2, bt2, g3, bt3):
    x2 = x[0]
    src = edge_index[0].reshape(NW, NBLK, NSUB, SUB)
    dst = edge_index[1].reshape(NW, NBLK, NSUB, SUB)
    ew = edge_weight

    wab = jnp.concatenate([W1[1], W1[2]], axis=1)
    wz = W1[0] - W1[2]
    wc2 = jnp.concatenate([W2[0] - W2[2], -W2[1], 2.0 * W2[2]], axis=0)
    wc3 = jnp.concatenate([W3[0] - W3[2], -W3[1], 2.0 * W3[2]], axis=0)
    b1r, b2r, b3r = b1[None, :], b2[None, :], b3[None, :]
    g1r, g2r, g3r = g1[None, :], g2[None, :], g3[None, :]
    bt1r, bt2r, bt3r = bt1[None, :], bt2[None, :], bt3[None, :]

    ones_tab = jnp.ones((N, 32), F32)
    degp = _sc_pass32(ones_tab, src, src, ew).reshape(NC, N, 32)
    dis, dis2, ab, z0 = _tc_proj1(degp, x2, wab, wz)

    up = _sc_pass64(ab, src, dst, ew).reshape(NC, N, 64)
    p_in = _tc_mid64(up, dis2)
    pp = _sc_pass32(p_in, src, dst, ew).reshape(NC, N, 32)
    y1, s3 = _tc_comb1(z0, up, pp, dis, b1r, g1r, bt1r)

    qp = _sc_pass32(s3, src, dst, ew).reshape(NC, N, 32)
    r_in = _tc_mid32(qp, dis2)
    rp = _sc_pass32(r_in, src, dst, ew).reshape(NC, N, 32)
    y2, s5 = _tc_comb2(y1, qp, rp, dis, wc2, b2r, g2r, bt2r)

    sp = _sc_pass32(s5, src, dst, ew).reshape(NC, N, 32)
    t_in = _tc_mid32(sp, dis2)
    tp = _sc_pass32(t_in, src, dst, ew).reshape(NC, N, 32)
    out = _tc_comb3(x2, y2, sp, tp, dis, wc3, b3r, g3r, bt3r)

    return out[None]


# speedup vs baseline: 14.7022x; 1.3041x over previous
import functools

import jax
import jax.numpy as jnp
from jax import lax
from jax.experimental import pallas as pl
from jax.experimental.pallas import tpu as pltpu
from jax.experimental.pallas import tpu_sc as plsc

N = 10000
E = 320000
NC = 2
NS = 16
EPT = E // NS
BLK = 400
SUB = 80
NSUB = BLK // SUB
NBLK = EPT // BLK
NPT = N // NS
DEPTH = 2
F32 = jnp.float32
I32 = jnp.int32

_MESH = plsc.VectorSubcoreMesh(
    core_axis_name="c", subcore_axis_name="s", num_cores=NC, num_subcores=NS)
_SC_PARAMS = pltpu.CompilerParams(use_tc_tiling_on_sc=False)


def _zero_rows(rows, nrows, ncols):
    def zr(i, _):
        for c0 in range(0, ncols, 16):
            rows[i, pl.ds(c0, 16)] = jnp.zeros((16,), F32)
        return 0
    lax.fori_loop(0, nrows, zr, 0)


_GDN = lax.GatherDimensionNumbers(
    offset_dims=(), collapsed_slice_dims=(0,), start_index_map=(0,))


def _bcast(v16, lane):
    idx = jnp.full((16, 1), lane, I32)
    return lax.gather(v16, idx, _GDN, (1,),
                      mode=lax.GatherScatterMode.PROMISE_IN_BOUNDS)


def _make_sc_pass(CH):
    @functools.partial(
        pl.kernel,
        out_type=jax.ShapeDtypeStruct((NC, NS, NPT, CH), F32),
        mesh=_MESH,
        compiler_params=_SC_PARAMS,
        scratch_types=[
            pltpu.VMEM((DEPTH, BLK, CH), F32),
            pltpu.VMEM((EPT // SUB, SUB), I32),
            pltpu.VMEM((EPT // SUB, SUB), I32),
            pltpu.VMEM((EPT,), F32),
            pltpu.VMEM_SHARED((N, CH), F32),
            pltpu.SemaphoreType.DMA,
            pltpu.SemaphoreType.DMA,
            pltpu.SemaphoreType.DMA,
            pltpu.SemaphoreType.DMA,
        ],
    )
    def _sc_pass(tab_hbm, src_hbm, dst_hbm, ew_hbm, out_hbm,
                 rows, srcv, dstv, eww, acc, gsem0, gsem1, ssem0, ssem1):
        c = lax.axis_index("c")
        s = lax.axis_index("s")
        tabc = tab_hbm.at[c]
        gsems = (gsem0, gsem1)
        ssems = (ssem0, ssem1)
        pltpu.sync_copy(src_hbm.at[s], srcv)
        pltpu.sync_copy(dst_hbm.at[s], dstv)
        pltpu.sync_copy(ew_hbm.at[pl.ds(s * EPT, EPT)], eww)
        _zero_rows(rows.at[0], BLK, CH)
        _zero_rows(rows.at[1], NPT - BLK, CH)
        base = s * NPT
        pltpu.sync_copy(rows.at[0], acc.at[pl.ds(base, BLK)])
        pltpu.sync_copy(rows.at[1, pl.ds(0, NPT - BLK)],
                        acc.at[pl.ds(base + BLK, NPT - BLK)])
        plsc.subcore_barrier()

        def fire_g(i, p):
            for t in range(NSUB):
                pltpu.async_copy(tabc.at[srcv.at[i * NSUB + t]],
                                 rows.at[p, pl.ds(t * SUB, SUB)], gsems[p])

        def wait_g(p):
            pltpu.make_async_copy(tabc.at[pl.ds(0, BLK)], rows.at[p],
                                  gsems[p]).wait()

        def scatter(i, p):
            cps = [
                pltpu.async_copy(rows.at[p, pl.ds(t * SUB, SUB)],
                                 acc.at[dstv.at[i * NSUB + t]], ssems[p],
                                 add=True)
                for t in range(NSUB)
            ]
            for cp in cps:
                cp.wait()

        def scale(i, p):
            def grp(g, _):
                e0 = g * 16
                w16 = eww[pl.ds(i * BLK + e0, 16)]
                for l in range(16):
                    wb = _bcast(w16, l)
                    for c0 in range(0, CH, 16):
                        v = rows[p, e0 + l, pl.ds(c0, 16)]
                        rows[p, e0 + l, pl.ds(c0, 16)] = v * wb
                return 0
            lax.fori_loop(0, BLK // 16, grp, 0)

        fire_g(0, 0)

        def pair(k, _):
            for b in range(2):
                i = 2 * k + b
                wait_g(b)
                scale(i, b)
                j = i + 1

                @pl.when(j < NBLK)
                def _():
                    fire_g(j, 1 - b)
                scatter(i, b)
            return 0
        lax.fori_loop(0, NBLK // 2, pair, 0)
        plsc.subcore_barrier()
        pltpu.sync_copy(acc.at[pl.ds(s * NPT, NPT)], out_hbm.at[c, s])
    return _sc_pass


_sc_pass32 = _make_sc_pass(32)
_sc_pass16 = _make_sc_pass(16)


def _proj1_body(degp, x, wab, wz, dis_o, dis2_o, ab_o, z0_o):
    deg = degp[0, :, 0:1]
    dis = jnp.where(deg > 0, lax.rsqrt(jnp.where(deg > 0, deg, 1.0)), 0.0)
    dis_o[...] = dis
    dis2_o[...] = dis * dis
    xx = x[...]
    ab = jnp.dot(xx, wab[...], preferred_element_type=F32) * dis
    ab_o[0] = ab[:, 0:32]
    ab_o[1] = ab[:, 32:64]
    z0_o[...] = jnp.dot(xx, wz[...], preferred_element_type=F32)


def _tc_proj1(degp, x2, wab, wz):
    return pl.pallas_call(
        _proj1_body,
        out_shape=[
            jax.ShapeDtypeStruct((N, 1), F32),
            jax.ShapeDtypeStruct((N, 1), F32),
            jax.ShapeDtypeStruct((NC, N, 32), F32),
            jax.ShapeDtypeStruct((N, 32), F32),
        ],
    )(degp, x2, wab, wz)


def _mid64_body(up, dis2, out):
    v = up[1] * dis2[...]
    out[0] = v[:, 0:16]
    out[1] = v[:, 16:32]


def _tc_mid64(up, dis2):
    return pl.pallas_call(
        _mid64_body,
        out_shape=jax.ShapeDtypeStruct((NC, N, 16), F32),
    )(up, dis2)


def _mid32_body(qp, dis2, out):
    out[0] = qp[0] * dis2[...]
    out[1] = qp[1] * dis2[...]


def _tc_mid32(qp, dis2):
    return pl.pallas_call(
        _mid32_body,
        out_shape=jax.ShapeDtypeStruct((NC, N, 16), F32),
    )(qp, dis2)


def _bn_relu(ypre, g, bt):
    m = jnp.mean(ypre, axis=0, keepdims=True)
    var = jnp.mean((ypre - m) ** 2, axis=0, keepdims=True)
    y = (ypre - m) * lax.rsqrt(var + 1e-5) * g + bt
    return jnp.maximum(y, 0.0)


def _halves(y, dis):
    yd = y * dis
    return yd[:, 0:16], yd[:, 16:32]


def _comb1_body(z0, up, pp, dis_r, b1, g1, bt1, y1_o, s3_o):
    dis = dis_r[...]
    u = up[0]
    p = jnp.concatenate([pp[0], pp[1]], axis=1)
    ypre = z0[...] - dis * u + 2.0 * dis * p + b1[...]
    y = _bn_relu(ypre, g1[...], bt1[...])
    y1_o[...] = y
    h0, h1 = _halves(y, dis)
    s3_o[0] = h0
    s3_o[1] = h1


def _tc_comb1(z0, up, pp, dis, b1, g1, bt1):
    return pl.pallas_call(
        _comb1_body,
        out_shape=[
            jax.ShapeDtypeStruct((N, 32), F32),
            jax.ShapeDtypeStruct((NC, N, 16), F32),
        ],
    )(z0, up, pp, dis, b1, g1, bt1)


def _comb2_body(y1, qp, rp, dis_r, wc, b, g, bt, y2_o, s5_o):
    dis = dis_r[...]
    q = jnp.concatenate([qp[0], qp[1]], axis=1)
    r = jnp.concatenate([rp[0], rp[1]], axis=1)
    h = jnp.concatenate([y1[...], dis * q, dis * r], axis=1)
    ypre = jnp.dot(h, wc[...], preferred_element_type=F32) + b[...]
    y = _bn_relu(ypre, g[...], bt[...])
    y2_o[...] = y
    h0, h1 = _halves(y, dis)
    s5_o[0] = h0
    s5_o[1] = h1


def _tc_comb2(y1, qp, rp, dis, wc, b, g, bt):
    return pl.pallas_call(
        _comb2_body,
        out_shape=[
            jax.ShapeDtypeStruct((N, 32), F32),
            jax.ShapeDtypeStruct((NC, N, 16), F32),
        ],
    )(y1, qp, rp, dis, wc, b, g, bt)


def _comb3_body(x, y2, sp, tp, dis_r, wc, b, g, bt, out_o):
    dis = dis_r[...]
    sv = jnp.concatenate([sp[0], sp[1]], axis=1)
    tv = jnp.concatenate([tp[0], tp[1]], axis=1)
    h = jnp.concatenate([y2[...], dis * sv, dis * tv], axis=1)
    ypre = jnp.dot(h, wc[...], preferred_element_type=F32) + b[...]
    y = _bn_relu(ypre, g[...], bt[...])
    out_o[...] = jnp.maximum(x[...] + y, 0.0)


def _tc_comb3(x2, y2, sp, tp, dis, wc, b, g, bt):
    return pl.pallas_call(
        _comb3_body,
        out_shape=jax.ShapeDtypeStruct((N, 128), F32),
    )(x2, y2, sp, tp, dis, wc, b, g, bt)


def kernel(x, edge_index, edge_weight, W1, b1, W2, b2, W3, b3,
           g1, bt1, g2, bt2, g3, bt3):
    x2 = x[0]
    src = edge_index[0].reshape(NS, EPT // SUB, SUB)
    dst = edge_index[1].reshape(NS, EPT // SUB, SUB)
    ew = edge_weight

    wab = jnp.concatenate([W1[1], W1[2]], axis=1)
    wz = W1[0] - W1[2]
    wc2 = jnp.concatenate([W2[0] - W2[2], -W2[1], 2.0 * W2[2]], axis=0)
    wc3 = jnp.concatenate([W3[0] - W3[2], -W3[1], 2.0 * W3[2]], axis=0)
    b1r, b2r, b3r = b1[None, :], b2[None, :], b3[None, :]
    g1r, g2r, g3r = g1[None, :], g2[None, :], g3[None, :]
    bt1r, bt2r, bt3r = bt1[None, :], bt2[None, :], bt3[None, :]

    ones_tab = jnp.ones((NC, N, 16), F32)
    degp = _sc_pass16(ones_tab, src, src, ew).reshape(NC, N, 16)
    dis, dis2, ab, z0 = _tc_proj1(degp, x2, wab, wz)

    up = _sc_pass32(ab, src, dst, ew).reshape(NC, N, 32)
    p_in = _tc_mid64(up, dis2)
    pp = _sc_pass16(p_in, src, dst, ew).reshape(NC, N, 16)
    y1, s3 = _tc_comb1(z0, up, pp, dis, b1r, g1r, bt1r)

    qp = _sc_pass16(s3, src, dst, ew).reshape(NC, N, 16)
    r_in = _tc_mid32(qp, dis2)
    rp = _sc_pass16(r_in, src, dst, ew).reshape(NC, N, 16)
    y2, s5 = _tc_comb2(y1, qp, rp, dis, wc2, b2r, g2r, bt2r)

    sp = _sc_pass16(s5, src, dst, ew).reshape(NC, N, 16)
    t_in = _tc_mid32(sp, dis2)
    tp = _sc_pass16(t_in, src, dst, ew).reshape(NC, N, 16)
    out = _tc_comb3(x2, y2, sp, tp, dis, wc3, b3r, g3r, bt3r)

    return out[None]


# speedup vs baseline: 18.3949x; 1.2512x over previous
import functools

import jax
import jax.numpy as jnp
from jax import lax
from jax.experimental import pallas as pl
from jax.experimental.pallas import tpu as pltpu
from jax.experimental.pallas import tpu_sc as plsc

N = 10000
E = 320000
NC = 2
NS = 16
EPT = E // NS
BLK = 400
SUB = 80
NSUB = BLK // SUB
NBLK = EPT // BLK
NPT = N // NS
DEPTH = 2
F32 = jnp.float32
I32 = jnp.int32

_MESH = plsc.VectorSubcoreMesh(
    core_axis_name="c", subcore_axis_name="s", num_cores=NC, num_subcores=NS)
_SC_PARAMS = pltpu.CompilerParams(use_tc_tiling_on_sc=False)


def _zero_rows(rows, nrows, ncols):
    def zr(i, _):
        for c0 in range(0, ncols, 16):
            rows[i, pl.ds(c0, 16)] = jnp.zeros((16,), F32)
        return 0
    lax.fori_loop(0, nrows, zr, 0)


_GDN = lax.GatherDimensionNumbers(
    offset_dims=(), collapsed_slice_dims=(0,), start_index_map=(0,))


def _bcast(v16, lane):
    idx = jnp.full((16, 1), lane, I32)
    return lax.gather(v16, idx, _GDN, (1,),
                      mode=lax.GatherScatterMode.PROMISE_IN_BOUNDS)


def _make_sc_pass(CH):
    @functools.partial(
        pl.kernel,
        out_type=jax.ShapeDtypeStruct((NC, NS, NPT, CH), F32),
        mesh=_MESH,
        compiler_params=_SC_PARAMS,
        scratch_types=[
            pltpu.VMEM((DEPTH, BLK, CH), F32),
            pltpu.VMEM((EPT // SUB, SUB), I32),
            pltpu.VMEM((EPT // SUB, SUB), I32),
            pltpu.VMEM((EPT,), F32),
            pltpu.VMEM_SHARED((N, CH), F32),
            pltpu.SemaphoreType.DMA,
            pltpu.SemaphoreType.DMA,
            pltpu.SemaphoreType.DMA,
            pltpu.SemaphoreType.DMA,
        ],
    )
    def _sc_pass(tab_hbm, src_hbm, dst_hbm, ew_hbm, out_hbm,
                 rows, srcv, dstv, eww, acc, gsem0, gsem1, ssem0, ssem1):
        c = lax.axis_index("c")
        s = lax.axis_index("s")
        tabc = tab_hbm.at[c]
        gsems = (gsem0, gsem1)
        ssems = (ssem0, ssem1)
        pltpu.sync_copy(src_hbm.at[s], srcv)
        pltpu.sync_copy(dst_hbm.at[s], dstv)
        pltpu.sync_copy(ew_hbm.at[pl.ds(s * EPT, EPT)], eww)
        _zero_rows(rows.at[0], BLK, CH)
        _zero_rows(rows.at[1], NPT - BLK, CH)
        base = s * NPT
        pltpu.sync_copy(rows.at[0], acc.at[pl.ds(base, BLK)])
        pltpu.sync_copy(rows.at[1, pl.ds(0, NPT - BLK)],
                        acc.at[pl.ds(base + BLK, NPT - BLK)])
        plsc.subcore_barrier()

        def fire_g(i, p):
            for t in range(NSUB):
                pltpu.async_copy(tabc.at[srcv.at[i * NSUB + t]],
                                 rows.at[p, pl.ds(t * SUB, SUB)], gsems[p])

        def wait_g(p):
            pltpu.make_async_copy(tabc.at[pl.ds(0, BLK)], rows.at[p],
                                  gsems[p]).wait()

        def fire_s(i, p):
            return [
                pltpu.async_copy(rows.at[p, pl.ds(t * SUB, SUB)],
                                 acc.at[dstv.at[i * NSUB + t]], ssems[p],
                                 add=True)
                for t in range(NSUB)
            ]

        def scale(i, p):
            def grp(g, _):
                e0 = g * 16
                w16 = eww[pl.ds(i * BLK + e0, 16)]
                for l in range(16):
                    wb = _bcast(w16, l)
                    for c0 in range(0, CH, 16):
                        v = rows[p, e0 + l, pl.ds(c0, 16)]
                        rows[p, e0 + l, pl.ds(c0, 16)] = v * wb
                return 0
            lax.fori_loop(0, BLK // 16, grp, 0)

        fire_g(0, 0)
        fire_g(1, 1)

        def pair(k, _):
            i0 = 2 * k
            i1 = i0 + 1
            wait_g(0)
            scale(i0, 0)
            cps0 = fire_s(i0, 0)
            wait_g(1)
            scale(i1, 1)
            cps1 = fire_s(i1, 1)
            for cp in cps0:
                cp.wait()

            @pl.when(i0 + 2 < NBLK)
            def _():
                fire_g(i0 + 2, 0)
            for cp in cps1:
                cp.wait()

            @pl.when(i1 + 2 < NBLK)
            def _():
                fire_g(i1 + 2, 1)
            return 0
        lax.fori_loop(0, NBLK // 2, pair, 0)
        plsc.subcore_barrier()
        pltpu.sync_copy(acc.at[pl.ds(s * NPT, NPT)], out_hbm.at[c, s])
    return _sc_pass


_sc_pass32 = _make_sc_pass(32)
_sc_pass16 = _make_sc_pass(16)


def _proj1_body(degp, x, wab, wz, dis_o, dis2_o, ab_o, z0_o):
    deg = degp[0, :, 0:1]
    dis = jnp.where(deg > 0, lax.rsqrt(jnp.where(deg > 0, deg, 1.0)), 0.0)
    dis_o[...] = dis
    dis2_o[...] = dis * dis
    xx = x[...]
    ab = jnp.dot(xx, wab[...], preferred_element_type=F32) * dis
    ab_o[0] = ab[:, 0:32]
    ab_o[1] = ab[:, 32:64]
    z0_o[...] = jnp.dot(xx, wz[...], preferred_element_type=F32)


def _tc_proj1(degp, x2, wab, wz):
    return pl.pallas_call(
        _proj1_body,
        out_shape=[
            jax.ShapeDtypeStruct((N, 1), F32),
            jax.ShapeDtypeStruct((N, 1), F32),
            jax.ShapeDtypeStruct((NC, N, 32), F32),
            jax.ShapeDtypeStruct((N, 32), F32),
        ],
    )(degp, x2, wab, wz)


def _mid64_body(up, dis2, out):
    v = up[1] * dis2[...]
    out[0] = v[:, 0:16]
    out[1] = v[:, 16:32]


def _tc_mid64(up, dis2):
    return pl.pallas_call(
        _mid64_body,
        out_shape=jax.ShapeDtypeStruct((NC, N, 16), F32),
    )(up, dis2)


def _mid32_body(qp, dis2, out):
    out[0] = qp[0] * dis2[...]
    out[1] = qp[1] * dis2[...]


def _tc_mid32(qp, dis2):
    return pl.pallas_call(
        _mid32_body,
        out_shape=jax.ShapeDtypeStruct((NC, N, 16), F32),
    )(qp, dis2)


def _bn_relu(ypre, g, bt):
    m = jnp.mean(ypre, axis=0, keepdims=True)
    var = jnp.mean((ypre - m) ** 2, axis=0, keepdims=True)
    y = (ypre - m) * lax.rsqrt(var + 1e-5) * g + bt
    return jnp.maximum(y, 0.0)


def _halves(y, dis):
    yd = y * dis
    return yd[:, 0:16], yd[:, 16:32]


def _comb1_body(z0, up, pp, dis_r, b1, g1, bt1, y1_o, s3_o):
    dis = dis_r[...]
    u = up[0]
    p = jnp.concatenate([pp[0], pp[1]], axis=1)
    ypre = z0[...] - dis * u + 2.0 * dis * p + b1[...]
    y = _bn_relu(ypre, g1[...], bt1[...])
    y1_o[...] = y
    h0, h1 = _halves(y, dis)
    s3_o[0] = h0
    s3_o[1] = h1


def _tc_comb1(z0, up, pp, dis, b1, g1, bt1):
    return pl.pallas_call(
        _comb1_body,
        out_shape=[
            jax.ShapeDtypeStruct((N, 32), F32),
            jax.ShapeDtypeStruct((NC, N, 16), F32),
        ],
    )(z0, up, pp, dis, b1, g1, bt1)


def _comb2_body(y1, qp, rp, dis_r, wc, b, g, bt, y2_o, s5_o):
    dis = dis_r[...]
    q = jnp.concatenate([qp[0], qp[1]], axis=1)
    r = jnp.concatenate([rp[0], rp[1]], axis=1)
    h = jnp.concatenate([y1[...], dis * q, dis * r], axis=1)
    ypre = jnp.dot(h, wc[...], preferred_element_type=F32) + b[...]
    y = _bn_relu(ypre, g[...], bt[...])
    y2_o[...] = y
    h0, h1 = _halves(y, dis)
    s5_o[0] = h0
    s5_o[1] = h1


def _tc_comb2(y1, qp, rp, dis, wc, b, g, bt):
    return pl.pallas_call(
        _comb2_body,
        out_shape=[
            jax.ShapeDtypeStruct((N, 32), F32),
            jax.ShapeDtypeStruct((NC, N, 16), F32),
        ],
    )(y1, qp, rp, dis, wc, b, g, bt)


def _comb3_body(x, y2, sp, tp, dis_r, wc, b, g, bt, out_o):
    dis = dis_r[...]
    sv = jnp.concatenate([sp[0], sp[1]], axis=1)
    tv = jnp.concatenate([tp[0], tp[1]], axis=1)
    h = jnp.concatenate([y2[...], dis * sv, dis * tv], axis=1)
    ypre = jnp.dot(h, wc[...], preferred_element_type=F32) + b[...]
    y = _bn_relu(ypre, g[...], bt[...])
    out_o[...] = jnp.maximum(x[...] + y, 0.0)


def _tc_comb3(x2, y2, sp, tp, dis, wc, b, g, bt):
    return pl.pallas_call(
        _comb3_body,
        out_shape=jax.ShapeDtypeStruct((N, 128), F32),
    )(x2, y2, sp, tp, dis, wc, b, g, bt)


def kernel(x, edge_index, edge_weight, W1, b1, W2, b2, W3, b3,
           g1, bt1, g2, bt2, g3, bt3):
    x2 = x[0]
    src = edge_index[0].reshape(NS, EPT // SUB, SUB)
    dst = edge_index[1].reshape(NS, EPT // SUB, SUB)
    ew = edge_weight

    wab = jnp.concatenate([W1[1], W1[2]], axis=1)
    wz = W1[0] - W1[2]
    wc2 = jnp.concatenate([W2[0] - W2[2], -W2[1], 2.0 * W2[2]], axis=0)
    wc3 = jnp.concatenate([W3[0] - W3[2], -W3[1], 2.0 * W3[2]], axis=0)
    b1r, b2r, b3r = b1[None, :], b2[None, :], b3[None, :]
    g1r, g2r, g3r = g1[None, :], g2[None, :], g3[None, :]
    bt1r, bt2r, bt3r = bt1[None, :], bt2[None, :], bt3[None, :]

    ones_tab = jnp.ones((NC, N, 16), F32)
    degp = _sc_pass16(ones_tab, src, src, ew).reshape(NC, N, 16)
    dis, dis2, ab, z0 = _tc_proj1(degp, x2, wab, wz)

    up = _sc_pass32(ab, src, dst, ew).reshape(NC, N, 32)
    p_in = _tc_mid64(up, dis2)
    pp = _sc_pass16(p_in, src, dst, ew).reshape(NC, N, 16)
    y1, s3 = _tc_comb1(z0, up, pp, dis, b1r, g1r, bt1r)

    qp = _sc_pass16(s3, src, dst, ew).reshape(NC, N, 16)
    r_in = _tc_mid32(qp, dis2)
    rp = _sc_pass16(r_in, src, dst, ew).reshape(NC, N, 16)
    y2, s5 = _tc_comb2(y1, qp, rp, dis, wc2, b2r, g2r, bt2r)

    sp = _sc_pass16(s5, src, dst, ew).reshape(NC, N, 16)
    t_in = _tc_mid32(sp, dis2)
    tp = _sc_pass16(t_in, src, dst, ew).reshape(NC, N, 16)
    out = _tc_comb3(x2, y2, sp, tp, dis, wc3, b3r, g3r, bt3r)

    return out[None]


# speedup vs baseline: 18.8122x; 1.0227x over previous
import functools

import jax
import jax.numpy as jnp
from jax import lax
from jax.experimental import pallas as pl
from jax.experimental.pallas import tpu as pltpu
from jax.experimental.pallas import tpu_sc as plsc

N = 10000
E = 320000
NC = 2
NS = 16
EPT = E // NS
BLK = 400
SUB = 80
NSUB = BLK // SUB
NBLK = EPT // BLK
NPT = N // NS
NCHK = 5
CHK = NPT // NCHK
DEPTH = 2
F32 = jnp.float32
I32 = jnp.int32

_MESH = plsc.VectorSubcoreMesh(
    core_axis_name="c", subcore_axis_name="s", num_cores=NC, num_subcores=NS)
_SC_PARAMS = pltpu.CompilerParams(use_tc_tiling_on_sc=False)


def _zero_rows(rows, nrows, ncols):
    def zr(i, _):
        for c0 in range(0, ncols, 16):
            rows[i, pl.ds(c0, 16)] = jnp.zeros((16,), F32)
        return 0
    lax.fori_loop(0, nrows, zr, 0)


_GDN = lax.GatherDimensionNumbers(
    offset_dims=(), collapsed_slice_dims=(0,), start_index_map=(0,))


def _bcast(v16, lane):
    idx = jnp.full((16, 1), lane, I32)
    return lax.gather(v16, idx, _GDN, (1,),
                      mode=lax.GatherScatterMode.PROMISE_IN_BOUNDS)


def _make_sc_pass(CH):
    @functools.partial(
        pl.kernel,
        out_type=[
            jax.ShapeDtypeStruct((NC, NS, NCHK, CHK, CH), F32),
            jax.ShapeDtypeStruct((NC, NS, NCHK, CHK, 16), F32),
        ],
        mesh=_MESH,
        compiler_params=_SC_PARAMS,
        scratch_types=[
            pltpu.VMEM((DEPTH, BLK, CH), F32),
            pltpu.VMEM((EPT // SUB, SUB), I32),
            pltpu.VMEM((EPT // SUB, SUB), I32),
            pltpu.VMEM((EPT,), F32),
            pltpu.VMEM((CHK, CH), F32),
            pltpu.VMEM((CHK, 16), F32),
            pltpu.VMEM((CHK, 16), F32),
            pltpu.VMEM_SHARED((N, CH), F32),
            pltpu.SemaphoreType.DMA,
            pltpu.SemaphoreType.DMA,
            pltpu.SemaphoreType.DMA,
            pltpu.SemaphoreType.DMA,
        ],
    )
    def _sc_pass(tab_hbm, src_hbm, dst_hbm, ew_hbm, d2_hbm, out_hbm, tab2_hbm,
                 rows, srcv, dstv, eww, vbuf, d2buf, sbuf, acc,
                 gsem0, gsem1, ssem0, ssem1):
        c = lax.axis_index("c")
        s = lax.axis_index("s")
        tabc = tab_hbm.at[c]
        gsems = (gsem0, gsem1)
        ssems = (ssem0, ssem1)
        pltpu.sync_copy(src_hbm.at[s], srcv)
        pltpu.sync_copy(dst_hbm.at[s], dstv)
        pltpu.sync_copy(ew_hbm.at[pl.ds(s * EPT, EPT)], eww)
        _zero_rows(rows.at[0], BLK, CH)
        _zero_rows(rows.at[1], NPT - BLK, CH)
        base = s * NPT
        pltpu.sync_copy(rows.at[0], acc.at[pl.ds(base, BLK)])
        pltpu.sync_copy(rows.at[1, pl.ds(0, NPT - BLK)],
                        acc.at[pl.ds(base + BLK, NPT - BLK)])
        plsc.subcore_barrier()

        def fire_g(i, p):
            for t in range(NSUB):
                pltpu.async_copy(tabc.at[srcv.at[i * NSUB + t]],
                                 rows.at[p, pl.ds(t * SUB, SUB)], gsems[p])

        def wait_g(p):
            pltpu.make_async_copy(tabc.at[pl.ds(0, BLK)], rows.at[p],
                                  gsems[p]).wait()

        def fire_s(i, p):
            return [
                pltpu.async_copy(rows.at[p, pl.ds(t * SUB, SUB)],
                                 acc.at[dstv.at[i * NSUB + t]], ssems[p],
                                 add=True)
                for t in range(NSUB)
            ]

        def scale(i, p):
            def grp(g, _):
                e0 = g * 16
                w16 = eww[pl.ds(i * BLK + e0, 16)]
                for l in range(16):
                    wb = _bcast(w16, l)
                    for c0 in range(0, CH, 16):
                        v = rows[p, e0 + l, pl.ds(c0, 16)]
                        rows[p, e0 + l, pl.ds(c0, 16)] = v * wb
                return 0
            lax.fori_loop(0, BLK // 16, grp, 0)

        fire_g(0, 0)
        fire_g(1, 1)

        def pair(k, _):
            i0 = 2 * k
            i1 = i0 + 1
            wait_g(0)
            scale(i0, 0)
            cps0 = fire_s(i0, 0)
            wait_g(1)
            scale(i1, 1)
            cps1 = fire_s(i1, 1)
            for cp in cps0:
                cp.wait()

            @pl.when(i0 + 2 < NBLK)
            def _():
                fire_g(i0 + 2, 0)
            for cp in cps1:
                cp.wait()

            @pl.when(i1 + 2 < NBLK)
            def _():
                fire_g(i1 + 2, 1)
            return 0
        lax.fori_loop(0, NBLK // 2, pair, 0)
        plsc.subcore_barrier()
        for ch in range(NCHK):
            pltpu.sync_copy(acc.at[pl.ds(base + ch * CHK, CHK)], vbuf)
            pltpu.sync_copy(d2_hbm.at[s, ch], d2buf)

            def rowscale(r, _):
                sbuf[r, pl.ds(0, 16)] = (vbuf[r, pl.ds(CH - 16, 16)]
                                         * d2buf[r, pl.ds(0, 16)])
                return 0
            lax.fori_loop(0, CHK, rowscale, 0)
            pltpu.sync_copy(vbuf, out_hbm.at[c, s, ch])
            pltpu.sync_copy(sbuf, tab2_hbm.at[c, s, ch])
    return _sc_pass


_sc_pass32 = _make_sc_pass(32)
_sc_pass16 = _make_sc_pass(16)


def _proj1_body(degp, x, wab, wz, dis_o, dis2_o, ab_o, z0_o):
    deg = degp[0, :, 0:1]
    dis = jnp.where(deg > 0, lax.rsqrt(jnp.where(deg > 0, deg, 1.0)), 0.0)
    dis_o[...] = dis
    dis2 = dis * dis
    dis2_o[...] = jnp.broadcast_to(dis2, (N, 16)).reshape(NS, NCHK, CHK, 16)
    xx = x[...]
    ab = jnp.dot(xx, wab[...], preferred_element_type=F32) * dis
    ab_o[0] = jnp.concatenate([ab[:, 0:16], ab[:, 32:48]], axis=1)
    ab_o[1] = jnp.concatenate([ab[:, 16:32], ab[:, 48:64]], axis=1)
    z0_o[...] = jnp.dot(xx, wz[...], preferred_element_type=F32)


def _tc_proj1(degp, x2, wab, wz):
    return pl.pallas_call(
        _proj1_body,
        out_shape=[
            jax.ShapeDtypeStruct((N, 1), F32),
            jax.ShapeDtypeStruct((NS, NCHK, CHK, 16), F32),
            jax.ShapeDtypeStruct((NC, N, 32), F32),
            jax.ShapeDtypeStruct((N, 32), F32),
        ],
    )(degp, x2, wab, wz)


def _bn_relu(ypre, g, bt):
    m = jnp.mean(ypre, axis=0, keepdims=True)
    var = jnp.mean((ypre - m) ** 2, axis=0, keepdims=True)
    y = (ypre - m) * lax.rsqrt(var + 1e-5) * g + bt
    return jnp.maximum(y, 0.0)


def _halves(y, dis):
    yd = y * dis
    return yd[:, 0:16], yd[:, 16:32]


def _comb1_body(z0, up, pp, dis_r, b1, g1, bt1, y1_o, s3_o):
    dis = dis_r[...]
    u = jnp.concatenate([up[0][:, 0:16], up[1][:, 0:16]], axis=1)
    p = jnp.concatenate([pp[0], pp[1]], axis=1)
    ypre = z0[...] - dis * u + 2.0 * dis * p + b1[...]
    y = _bn_relu(ypre, g1[...], bt1[...])
    y1_o[...] = y
    h0, h1 = _halves(y, dis)
    s3_o[0] = h0
    s3_o[1] = h1


def _tc_comb1(z0, up, pp, dis, b1, g1, bt1):
    return pl.pallas_call(
        _comb1_body,
        out_shape=[
            jax.ShapeDtypeStruct((N, 32), F32),
            jax.ShapeDtypeStruct((NC, N, 16), F32),
        ],
    )(z0, up, pp, dis, b1, g1, bt1)


def _comb2_body(y1, qp, rp, dis_r, wc, b, g, bt, y2_o, s5_o):
    dis = dis_r[...]
    q = jnp.concatenate([qp[0], qp[1]], axis=1)
    r = jnp.concatenate([rp[0], rp[1]], axis=1)
    h = jnp.concatenate([y1[...], dis * q, dis * r], axis=1)
    ypre = jnp.dot(h, wc[...], preferred_element_type=F32) + b[...]
    y = _bn_relu(ypre, g[...], bt[...])
    y2_o[...] = y
    h0, h1 = _halves(y, dis)
    s5_o[0] = h0
    s5_o[1] = h1


def _tc_comb2(y1, qp, rp, dis, wc, b, g, bt):
    return pl.pallas_call(
        _comb2_body,
        out_shape=[
            jax.ShapeDtypeStruct((N, 32), F32),
            jax.ShapeDtypeStruct((NC, N, 16), F32),
        ],
    )(y1, qp, rp, dis, wc, b, g, bt)


def _comb3_body(x, y2, sp, tp, dis_r, wc, b, g, bt, out_o):
    dis = dis_r[...]
    sv = jnp.concatenate([sp[0], sp[1]], axis=1)
    tv = jnp.concatenate([tp[0], tp[1]], axis=1)
    h = jnp.concatenate([y2[...], dis * sv, dis * tv], axis=1)
    ypre = jnp.dot(h, wc[...], preferred_element_type=F32) + b[...]
    y = _bn_relu(ypre, g[...], bt[...])
    out_o[...] = jnp.maximum(x[...] + y, 0.0)


def _tc_comb3(x2, y2, sp, tp, dis, wc, b, g, bt):
    return pl.pallas_call(
        _comb3_body,
        out_shape=jax.ShapeDtypeStruct((N, 128), F32),
    )(x2, y2, sp, tp, dis, wc, b, g, bt)


def kernel(x, edge_index, edge_weight, W1, b1, W2, b2, W3, b3,
           g1, bt1, g2, bt2, g3, bt3):
    x2 = x[0]
    src = edge_index[0].reshape(NS, EPT // SUB, SUB)
    dst = edge_index[1].reshape(NS, EPT // SUB, SUB)
    ew = edge_weight

    wab = jnp.concatenate([W1[1], W1[2]], axis=1)
    wz = W1[0] - W1[2]
    wc2 = jnp.concatenate([W2[0] - W2[2], -W2[1], 2.0 * W2[2]], axis=0)
    wc3 = jnp.concatenate([W3[0] - W3[2], -W3[1], 2.0 * W3[2]], axis=0)
    b1r, b2r, b3r = b1[None, :], b2[None, :], b3[None, :]
    g1r, g2r, g3r = g1[None, :], g2[None, :], g3[None, :]
    bt1r, bt2r, bt3r = bt1[None, :], bt2[None, :], bt3[None, :]

    ones_tab = jnp.ones((NC, N, 16), F32)
    ones_d2 = jnp.ones((NS, NCHK, CHK, 16), F32)
    degp, _ = _sc_pass16(ones_tab, src, src, ew, ones_d2)
    degp = degp.reshape(NC, N, 16)
    dis, dis2, ab, z0 = _tc_proj1(degp, x2, wab, wz)

    up, p_in = _sc_pass32(ab, src, dst, ew, dis2)
    up = up.reshape(NC, N, 32)
    pp, _ = _sc_pass16(p_in.reshape(NC, N, 16), src, dst, ew, dis2)
    pp = pp.reshape(NC, N, 16)
    y1, s3 = _tc_comb1(z0, up, pp, dis, b1r, g1r, bt1r)

    qp, r_in = _sc_pass16(s3, src, dst, ew, dis2)
    qp = qp.reshape(NC, N, 16)
    rp, _ = _sc_pass16(r_in.reshape(NC, N, 16), src, dst, ew, dis2)
    rp = rp.reshape(NC, N, 16)
    y2, s5 = _tc_comb2(y1, qp, rp, dis, wc2, b2r, g2r, bt2r)

    sp, t_in = _sc_pass16(s5, src, dst, ew, dis2)
    sp = sp.reshape(NC, N, 16)
    tp, _ = _sc_pass16(t_in.reshape(NC, N, 16), src, dst, ew, dis2)
    tp = tp.reshape(NC, N, 16)
    out = _tc_comb3(x2, y2, sp, tp, dis, wc3, b3r, g3r, bt3r)

    return out[None]


# speedup vs baseline: 19.8357x; 1.0544x over previous
import functools

import jax
import jax.numpy as jnp
from jax import lax
from jax.experimental import pallas as pl
from jax.experimental.pallas import tpu as pltpu
from jax.experimental.pallas import tpu_sc as plsc

N = 10000
E = 320000
NC = 2
NS = 16
EPT = E // NS
BLK = 400
SUB = 80
NSUB = BLK // SUB
NBLK = EPT // BLK
NPT = N // NS
NCHK = 5
CHK = NPT // NCHK
DEPTH = 2
F32 = jnp.float32
I32 = jnp.int32

_MESH = plsc.VectorSubcoreMesh(
    core_axis_name="c", subcore_axis_name="s", num_cores=NC, num_subcores=NS)
_SC_PARAMS = pltpu.CompilerParams(use_tc_tiling_on_sc=False)


def _zero_rows(rows, nrows, ncols):
    def zr(i, _):
        for c0 in range(0, ncols, 16):
            rows[i, pl.ds(c0, 16)] = jnp.zeros((16,), F32)
        return 0
    lax.fori_loop(0, nrows, zr, 0)


_GDN = lax.GatherDimensionNumbers(
    offset_dims=(), collapsed_slice_dims=(0,), start_index_map=(0,))


def _bcast(v16, lane):
    idx = jnp.full((16, 1), lane, I32)
    return lax.gather(v16, idx, _GDN, (1,),
                      mode=lax.GatherScatterMode.PROMISE_IN_BOUNDS)


def _make_sc_pass(CH):
    @functools.partial(
        pl.kernel,
        out_type=[
            jax.ShapeDtypeStruct((NC, NS, NCHK, CHK, CH), F32),
            jax.ShapeDtypeStruct((NC, NS, NCHK, CHK, 16), F32),
        ],
        mesh=_MESH,
        compiler_params=_SC_PARAMS,
        scratch_types=[
            pltpu.VMEM((DEPTH, BLK, CH), F32),
            pltpu.VMEM((EPT // SUB, SUB), I32),
            pltpu.VMEM((EPT // SUB, SUB), I32),
            pltpu.VMEM((EPT,), F32),
            pltpu.VMEM((CHK, CH), F32),
            pltpu.VMEM((CHK, 16), F32),
            pltpu.VMEM((CHK, 16), F32),
            pltpu.VMEM_SHARED((N, CH), F32),
            pltpu.SemaphoreType.DMA,
            pltpu.SemaphoreType.DMA,
            pltpu.SemaphoreType.DMA,
            pltpu.SemaphoreType.DMA,
        ],
    )
    def _sc_pass(tab_hbm, src_hbm, dst_hbm, ew_hbm, d2_hbm, out_hbm, tab2_hbm,
                 rows, srcv, dstv, eww, vbuf, d2buf, sbuf, acc,
                 gsem0, gsem1, ssem0, ssem1):
        c = lax.axis_index("c")
        s = lax.axis_index("s")
        tabc = tab_hbm.at[c]
        gsems = (gsem0, gsem1)
        ssems = (ssem0, ssem1)
        pltpu.sync_copy(src_hbm.at[s], srcv)
        pltpu.sync_copy(dst_hbm.at[s], dstv)
        pltpu.sync_copy(ew_hbm.at[pl.ds(s * EPT, EPT)], eww)
        _zero_rows(rows.at[0], BLK, CH)
        _zero_rows(rows.at[1], NPT - BLK, CH)
        base = s * NPT
        pltpu.sync_copy(rows.at[0], acc.at[pl.ds(base, BLK)])
        pltpu.sync_copy(rows.at[1, pl.ds(0, NPT - BLK)],
                        acc.at[pl.ds(base + BLK, NPT - BLK)])
        plsc.subcore_barrier()

        def fire_g(i, p):
            for t in range(NSUB):
                pltpu.async_copy(tabc.at[srcv.at[i * NSUB + t]],
                                 rows.at[p, pl.ds(t * SUB, SUB)], gsems[p])

        def wait_g(p):
            pltpu.make_async_copy(tabc.at[pl.ds(0, BLK)], rows.at[p],
                                  gsems[p]).wait()

        def fire_s(i, p):
            return [
                pltpu.async_copy(rows.at[p, pl.ds(t * SUB, SUB)],
                                 acc.at[dstv.at[i * NSUB + t]], ssems[p],
                                 add=True)
                for t in range(NSUB)
            ]

        def scale(i, p):
            def grp(g, _):
                e0 = g * 16
                w16 = eww[pl.ds(i * BLK + e0, 16)]
                for l in range(16):
                    wb = _bcast(w16, l)
                    for c0 in range(0, CH, 16):
                        v = rows[p, e0 + l, pl.ds(c0, 16)]
                        rows[p, e0 + l, pl.ds(c0, 16)] = v * wb
                return 0
            lax.fori_loop(0, BLK // 16, grp, 0)

        fire_g(0, 0)
        fire_g(1, 1)

        def pair(k, _):
            i0 = 2 * k
            i1 = i0 + 1
            wait_g(0)
            scale(i0, 0)
            cps0 = fire_s(i0, 0)
            wait_g(1)
            scale(i1, 1)
            cps1 = fire_s(i1, 1)
            for cp in cps0:
                cp.wait()

            @pl.when(i0 + 2 < NBLK)
            def _():
                fire_g(i0 + 2, 0)
            for cp in cps1:
                cp.wait()

            @pl.when(i1 + 2 < NBLK)
            def _():
                fire_g(i1 + 2, 1)
            return 0
        lax.fori_loop(0, NBLK // 2, pair, 0)
        plsc.subcore_barrier()
        for ch in range(NCHK):
            pltpu.sync_copy(acc.at[pl.ds(base + ch * CHK, CHK)], vbuf)
            pltpu.sync_copy(d2_hbm.at[s, ch], d2buf)

            def rowscale(r, _):
                sbuf[r, pl.ds(0, 16)] = (vbuf[r, pl.ds(CH - 16, 16)]
                                         * d2buf[r, pl.ds(0, 16)])
                return 0
            lax.fori_loop(0, CHK, rowscale, 0)
            pltpu.sync_copy(vbuf, out_hbm.at[c, s, ch])
            pltpu.sync_copy(sbuf, tab2_hbm.at[c, s, ch])
    return _sc_pass


_sc_pass16 = _make_sc_pass(16)
_sc_pass32 = _make_sc_pass(32)


def _make_sc_merged(CH):
    @functools.partial(
        pl.kernel,
        out_type=[
            jax.ShapeDtypeStruct((NC, NS, NCHK, CHK, 16), F32),
            jax.ShapeDtypeStruct((NC, NS, NCHK, CHK, 16), F32),
        ],
        mesh=_MESH,
        compiler_params=_SC_PARAMS,
        scratch_types=[
            pltpu.VMEM((DEPTH, BLK, CH), F32),
            pltpu.VMEM((DEPTH, BLK, 16), F32),
            pltpu.VMEM((EPT // SUB, SUB), I32),
            pltpu.VMEM((EPT // SUB, SUB), I32),
            pltpu.VMEM((EPT,), F32),
            pltpu.VMEM((CHK, CH), F32),
            pltpu.VMEM((CHK, 16), F32),
            pltpu.VMEM((CHK, 16), F32),
            pltpu.VMEM((CHK, 16), F32),
            pltpu.VMEM_SHARED((N, CH), F32),
            pltpu.VMEM_SHARED((N, 16) if CH > 16 else (16, 16), F32),
            pltpu.VMEM_SHARED((N, 16), F32),
            pltpu.SemaphoreType.DMA,
            pltpu.SemaphoreType.DMA,
            pltpu.SemaphoreType.DMA,
            pltpu.SemaphoreType.DMA,
        ],
    )
    def _sc_merged(tab_hbm, src_hbm, dst_hbm, ew_hbm, d2_hbm,
                   out1_hbm, out2_hbm,
                   rows_a, rows_b, srcv, dstv, eww, vbuf, d2buf, sbuf, ubuf,
                   acc1, tab2, acc2, gsem0, gsem1, ssem0, ssem1):
        c = lax.axis_index("c")
        s = lax.axis_index("s")
        gsems = (gsem0, gsem1)
        ssems = (ssem0, ssem1)
        pltpu.sync_copy(src_hbm.at[s], srcv)
        pltpu.sync_copy(dst_hbm.at[s], dstv)
        pltpu.sync_copy(ew_hbm.at[pl.ds(s * EPT, EPT)], eww)
        base = s * NPT
        _zero_rows(rows_a.at[0], BLK, CH)
        _zero_rows(rows_a.at[1], NPT - BLK, CH)
        pltpu.sync_copy(rows_a.at[0], acc1.at[pl.ds(base, BLK)])
        pltpu.sync_copy(rows_a.at[1, pl.ds(0, NPT - BLK)],
                        acc1.at[pl.ds(base + BLK, NPT - BLK)])
        _zero_rows(rows_b.at[0], BLK, 16)
        _zero_rows(rows_b.at[1], NPT - BLK, 16)
        pltpu.sync_copy(rows_b.at[0], acc2.at[pl.ds(base, BLK)])
        pltpu.sync_copy(rows_b.at[1, pl.ds(0, NPT - BLK)],
                        acc2.at[pl.ds(base + BLK, NPT - BLK)])
        plsc.subcore_barrier()

        def run_phase(tabc, rows, acc, chw):
            def fire_g(i, p):
                for t in range(NSUB):
                    pltpu.async_copy(tabc.at[srcv.at[i * NSUB + t]],
                                     rows.at[p, pl.ds(t * SUB, SUB)],
                                     gsems[p])

            def wait_g(p):
                pltpu.make_async_copy(tabc.at[pl.ds(0, BLK)], rows.at[p],
                                      gsems[p]).wait()

            def fire_s(i, p):
                return [
                    pltpu.async_copy(rows.at[p, pl.ds(t * SUB, SUB)],
                                     acc.at[dstv.at[i * NSUB + t]], ssems[p],
                                     add=True)
                    for t in range(NSUB)
                ]

            def scale(i, p):
                def grp(g, _):
                    e0 = g * 16
                    w16 = eww[pl.ds(i * BLK + e0, 16)]
                    for l in range(16):
                        wb = _bcast(w16, l)
                        for c0 in range(0, chw, 16):
                            v = rows[p, e0 + l, pl.ds(c0, 16)]
                            rows[p, e0 + l, pl.ds(c0, 16)] = v * wb
                    return 0
                lax.fori_loop(0, BLK // 16, grp, 0)

            fire_g(0, 0)
            fire_g(1, 1)

            def pair(k, _):
                i0 = 2 * k
                i1 = i0 + 1
                wait_g(0)
                scale(i0, 0)
                cps0 = fire_s(i0, 0)
                wait_g(1)
                scale(i1, 1)
                cps1 = fire_s(i1, 1)
                for cp in cps0:
                    cp.wait()

                @pl.when(i0 + 2 < NBLK)
                def _():
                    fire_g(i0 + 2, 0)
                for cp in cps1:
                    cp.wait()

                @pl.when(i1 + 2 < NBLK)
                def _():
                    fire_g(i1 + 2, 1)
                return 0
            lax.fori_loop(0, NBLK // 2, pair, 0)

        run_phase(tab_hbm.at[c], rows_a, acc1, CH)
        plsc.subcore_barrier()
        tab2_ref = tab2 if CH > 16 else acc1
        for ch in range(NCHK):
            pltpu.sync_copy(acc1.at[pl.ds(base + ch * CHK, CHK)], vbuf)
            pltpu.sync_copy(d2_hbm.at[s, ch], d2buf)

            def rowscale(r, _):
                sbuf[r, pl.ds(0, 16)] = (vbuf[r, pl.ds(CH - 16, 16)]
                                         * d2buf[r, pl.ds(0, 16)])
                ubuf[r, pl.ds(0, 16)] = vbuf[r, pl.ds(0, 16)]
                return 0
            lax.fori_loop(0, CHK, rowscale, 0)
            pltpu.sync_copy(ubuf, out1_hbm.at[c, s, ch])
            pltpu.sync_copy(sbuf, tab2_ref.at[pl.ds(base + ch * CHK, CHK)])
        plsc.subcore_barrier()
        run_phase(tab2_ref, rows_b, acc2, 16)
        plsc.subcore_barrier()
        for ch in range(NCHK):
            pltpu.sync_copy(acc2.at[pl.ds(base + ch * CHK, CHK)],
                            out2_hbm.at[c, s, ch])
    return _sc_merged


_sc_merged16 = _make_sc_merged(16)


def _proj1_body(degp, x, wab, wz, dis_o, dis2_o, ab_o, z0_o):
    deg = degp[0, :, 0:1]
    dis = jnp.where(deg > 0, lax.rsqrt(jnp.where(deg > 0, deg, 1.0)), 0.0)
    dis_o[...] = dis
    dis2 = dis * dis
    dis2_o[...] = jnp.broadcast_to(dis2, (N, 16)).reshape(NS, NCHK, CHK, 16)
    xx = x[...]
    ab = jnp.dot(xx, wab[...], preferred_element_type=F32) * dis
    ab_o[0] = jnp.concatenate([ab[:, 0:16], ab[:, 32:48]], axis=1)
    ab_o[1] = jnp.concatenate([ab[:, 16:32], ab[:, 48:64]], axis=1)
    z0_o[...] = jnp.dot(xx, wz[...], preferred_element_type=F32)


def _tc_proj1(degp, x2, wab, wz):
    return pl.pallas_call(
        _proj1_body,
        out_shape=[
            jax.ShapeDtypeStruct((N, 1), F32),
            jax.ShapeDtypeStruct((NS, NCHK, CHK, 16), F32),
            jax.ShapeDtypeStruct((NC, N, 32), F32),
            jax.ShapeDtypeStruct((N, 32), F32),
        ],
    )(degp, x2, wab, wz)


def _bn_relu(ypre, g, bt):
    m = jnp.mean(ypre, axis=0, keepdims=True)
    var = jnp.mean((ypre - m) ** 2, axis=0, keepdims=True)
    y = (ypre - m) * lax.rsqrt(var + 1e-5) * g + bt
    return jnp.maximum(y, 0.0)


def _halves(y, dis):
    yd = y * dis
    return yd[:, 0:16], yd[:, 16:32]


def _comb1_body(z0, up, pp, dis_r, b1, g1, bt1, y1_o, s3_o):
    dis = dis_r[...]
    u = jnp.concatenate([up[0][:, 0:16], up[1][:, 0:16]], axis=1)
    p = jnp.concatenate([pp[0], pp[1]], axis=1)
    ypre = z0[...] - dis * u + 2.0 * dis * p + b1[...]
    y = _bn_relu(ypre, g1[...], bt1[...])
    y1_o[...] = y
    h0, h1 = _halves(y, dis)
    s3_o[0] = h0
    s3_o[1] = h1


def _tc_comb1(z0, up, pp, dis, b1, g1, bt1):
    return pl.pallas_call(
        _comb1_body,
        out_shape=[
            jax.ShapeDtypeStruct((N, 32), F32),
            jax.ShapeDtypeStruct((NC, N, 16), F32),
        ],
    )(z0, up, pp, dis, b1, g1, bt1)


def _comb2_body(y1, qp, rp, dis_r, wc, b, g, bt, y2_o, s5_o):
    dis = dis_r[...]
    q = jnp.concatenate([qp[0], qp[1]], axis=1)
    r = jnp.concatenate([rp[0], rp[1]], axis=1)
    h = jnp.concatenate([y1[...], dis * q, dis * r], axis=1)
    ypre = jnp.dot(h, wc[...], preferred_element_type=F32) + b[...]
    y = _bn_relu(ypre, g[...], bt[...])
    y2_o[...] = y
    h0, h1 = _halves(y, dis)
    s5_o[0] = h0
    s5_o[1] = h1


def _tc_comb2(y1, qp, rp, dis, wc, b, g, bt):
    return pl.pallas_call(
        _comb2_body,
        out_shape=[
            jax.ShapeDtypeStruct((N, 32), F32),
            jax.ShapeDtypeStruct((NC, N, 16), F32),
        ],
    )(y1, qp, rp, dis, wc, b, g, bt)


def _comb3_body(x, y2, sp, tp, dis_r, wc, b, g, bt, out_o):
    dis = dis_r[...]
    sv = jnp.concatenate([sp[0], sp[1]], axis=1)
    tv = jnp.concatenate([tp[0], tp[1]], axis=1)
    h = jnp.concatenate([y2[...], dis * sv, dis * tv], axis=1)
    ypre = jnp.dot(h, wc[...], preferred_element_type=F32) + b[...]
    y = _bn_relu(ypre, g[...], bt[...])
    out_o[...] = jnp.maximum(x[...] + y, 0.0)


def _tc_comb3(x2, y2, sp, tp, dis, wc, b, g, bt):
    return pl.pallas_call(
        _comb3_body,
        out_shape=jax.ShapeDtypeStruct((N, 128), F32),
    )(x2, y2, sp, tp, dis, wc, b, g, bt)


def kernel(x, edge_index, edge_weight, W1, b1, W2, b2, W3, b3,
           g1, bt1, g2, bt2, g3, bt3):
    x2 = x[0]
    src = edge_index[0].reshape(NS, EPT // SUB, SUB)
    dst = edge_index[1].reshape(NS, EPT // SUB, SUB)
    ew = edge_weight

    wab = jnp.concatenate([W1[1], W1[2]], axis=1)
    wz = W1[0] - W1[2]
    wc2 = jnp.concatenate([W2[0] - W2[2], -W2[1], 2.0 * W2[2]], axis=0)
    wc3 = jnp.concatenate([W3[0] - W3[2], -W3[1], 2.0 * W3[2]], axis=0)
    b1r, b2r, b3r = b1[None, :], b2[None, :], b3[None, :]
    g1r, g2r, g3r = g1[None, :], g2[None, :], g3[None, :]
    bt1r, bt2r, bt3r = bt1[None, :], bt2[None, :], bt3[None, :]

    ones_tab = jnp.ones((NC, N, 16), F32)
    ones_d2 = jnp.ones((NS, NCHK, CHK, 16), F32)
    degp, _ = _sc_pass16(ones_tab, src, src, ew, ones_d2)
    degp = degp.reshape(NC, N, 16)
    dis, dis2, ab, z0 = _tc_proj1(degp, x2, wab, wz)

    up, p_in = _sc_pass32(ab, src, dst, ew, dis2)
    up = up.reshape(NC, N, 32)
    pp, _ = _sc_pass16(p_in.reshape(NC, N, 16), src, dst, ew, dis2)
    pp = pp.reshape(NC, N, 16)
    y1, s3 = _tc_comb1(z0, up, pp, dis, b1r, g1r, bt1r)

    qp, rp = _sc_merged16(s3, src, dst, ew, dis2)
    qp = qp.reshape(NC, N, 16)
    rp = rp.reshape(NC, N, 16)
    y2, s5 = _tc_comb2(y1, qp, rp, dis, wc2, b2r, g2r, bt2r)

    sp, tp = _sc_merged16(s5, src, dst, ew, dis2)
    sp = sp.reshape(NC, N, 16)
    tp = tp.reshape(NC, N, 16)
    out = _tc_comb3(x2, y2, sp, tp, dis, wc3, b3r, g3r, bt3r)

    return out[None]
